# Initial kernel scaffold; baseline (speedup 1.0000x reference)
import jax
import jax.numpy as jnp
from jax.experimental import pallas as pl


def kernel(x, edge_index, W1, b1, W2, b2, Wc1, bc1, Wc2, bc2, Wc3, bc3):
    raise NotImplementedError("write your pallas kernel here")


# speedup vs baseline: 2.5115x; 2.5115x over previous
import functools

import jax
import jax.numpy as jnp
from jax import lax
from jax.experimental import pallas as pl
from jax.experimental.pallas import tpu as pltpu
from jax.experimental.pallas import tpu_sc as plsc

NC = 2
NS = 16
L = 16
NW = NC * NS

F32 = jnp.float32


def _mesh():
    return plsc.VectorSubcoreMesh(
        core_axis_name="c", subcore_axis_name="s", num_cores=NC, num_subcores=NS
    )


def _degrees(src, dst, n_nodes):
    e = src.shape[0]
    ept = e // NW

    @functools.partial(
        pl.kernel,
        out_type=jax.ShapeDtypeStruct((2 * NW, n_nodes), F32),
        mesh=_mesh(),
        compiler_params=pltpu.CompilerParams(needs_layout_passes=False),
        scratch_types=[
            pltpu.VMEM((ept,), jnp.int32),
            pltpu.VMEM((ept,), jnp.int32),
            pltpu.VMEM((n_nodes,), F32),
            pltpu.VMEM((n_nodes,), F32),
        ],
    )
    def deg_kernel(src_hbm, dst_hbm, out_hbm, s_v, d_v, od_v, id_v):
        wid = lax.axis_index("s") * NC + lax.axis_index("c")
        base = wid * ept
        pltpu.sync_copy(src_hbm.at[pl.ds(base, ept)], s_v)
        pltpu.sync_copy(dst_hbm.at[pl.ds(base, ept)], d_v)

        zero = jnp.zeros((L,), F32)

        def zbody(i, _):
            od_v[pl.ds(i * L, L)] = zero
            id_v[pl.ds(i * L, L)] = zero
            return ()

        lax.fori_loop(0, n_nodes // L, zbody, ())

        ones = jnp.ones((L,), F32)

        def ebody(g, _):
            s16 = s_v[pl.ds(g * L, L)]
            d16 = d_v[pl.ds(g * L, L)]
            plsc.addupdate_scatter(od_v, [s16], ones)
            plsc.addupdate_scatter(id_v, [d16], ones)
            return ()

        lax.fori_loop(0, ept // L, ebody, ())

        pltpu.sync_copy(od_v, out_hbm.at[wid])
        pltpu.sync_copy(id_v, out_hbm.at[NW + wid])

    return deg_kernel(src, dst)


def _norms(deg_part, n_nodes):
    def body(dp_ref, out_ref):
        od = jnp.sum(dp_ref[0:NW, :], axis=0, keepdims=True)
        idg = jnp.sum(dp_ref[NW : 2 * NW, :], axis=0, keepdims=True)
        ns = lax.rsqrt(jnp.maximum(od, 1.0))
        nd = lax.rsqrt(jnp.maximum(idg, 1.0))
        out_ref[...] = jnp.concatenate([ns, nd], axis=0)

    return pl.pallas_call(
        body,
        out_shape=jax.ShapeDtypeStruct((2, n_nodes), F32),
    )(deg_part)


def _mm1_scaled(x, w, norm2):
    n, f = x.shape

    def body(x_ref, w_ref, nrm_ref, out_ref):
        mm = lax.dot_general(
            w_ref[...], x_ref[...], (((0,), (1,)), ((), ())),
            preferred_element_type=F32,
        )
        out_ref[...] = mm * nrm_ref[0:1, :]

    return pl.pallas_call(
        body,
        out_shape=jax.ShapeDtypeStruct((f, n), F32),
    )(x, w, norm2)


def _layer2_scaled(aggT, w, b2d, norm2):
    f, n = aggT.shape

    def body(agg_ref, w_ref, b_ref, nrm_ref, out_ref):
        h = jnp.maximum(agg_ref[...] * nrm_ref[1:2, :] + b_ref[...], 0.0)
        mm = lax.dot_general(
            w_ref[...], h, (((0,), (0,)), ((), ())),
            preferred_element_type=F32,
        )
        out_ref[...] = mm * nrm_ref[0:1, :]

    return pl.pallas_call(
        body,
        out_shape=jax.ShapeDtypeStruct((f, n), F32),
    )(aggT, w, b2d, norm2)


def _edge_pass(mT, src, dst):
    f, n = mT.shape
    e = src.shape[0]
    fpt = f // NW
    chunk = 3200
    nch = e // chunk

    @functools.partial(
        pl.kernel,
        out_type=jax.ShapeDtypeStruct((f, n), F32),
        mesh=_mesh(),
        compiler_params=pltpu.CompilerParams(needs_layout_passes=False),
        scratch_types=[
            pltpu.VMEM((fpt, n), F32),
            pltpu.VMEM((fpt, n), F32),
            pltpu.VMEM((chunk,), jnp.int32),
            pltpu.VMEM((chunk,), jnp.int32),
        ],
    )
    def edge_kernel(mT_hbm, src_hbm, dst_hbm, aggT_hbm, m_v, agg_v, s_v, d_v):
        wid = lax.axis_index("s") * NC + lax.axis_index("c")
        fbase = wid * fpt
        pltpu.sync_copy(mT_hbm.at[pl.ds(fbase, fpt), :], m_v)

        zero = jnp.zeros((L,), F32)

        def zbody(i, _):
            for j in range(fpt):
                agg_v[j, pl.ds(i * L, L)] = zero
            return ()

        lax.fori_loop(0, n // L, zbody, ())

        jidx = [jnp.full((L,), j, jnp.int32) for j in range(fpt)]

        def cbody(c, _):
            pltpu.sync_copy(src_hbm.at[pl.ds(c * chunk, chunk)], s_v)
            pltpu.sync_copy(dst_hbm.at[pl.ds(c * chunk, chunk)], d_v)

            def ebody(g, _):
                s16 = s_v[pl.ds(g * L, L)]
                d16 = d_v[pl.ds(g * L, L)]
                for j in range(fpt):
                    vals = plsc.load_gather(m_v, [jidx[j], s16])
                    plsc.addupdate_scatter(agg_v, [jidx[j], d16], vals)
                return ()

            lax.fori_loop(0, chunk // L, ebody, ())
            return ()

        lax.fori_loop(0, nch, cbody, ())

        pltpu.sync_copy(agg_v, aggT_hbm.at[pl.ds(fbase, fpt), :])

    return edge_kernel(mT, src, dst)


def _head(aggT, norm2, b2d, wc1, bc1, wc2, bc2, wc3, bc3):
    f, n = aggT.shape

    def body(agg_ref, nrm_ref, b_ref, wc1_ref, bc1_ref, wc2_ref, bc2_ref,
             wc3_ref, bc3_ref, out_ref):
        h = jnp.maximum(agg_ref[...] * nrm_ref[1:2, :] + b_ref[...], 0.0)
        hg = jnp.sum(h, axis=1, keepdims=True) * (1.0 / n)
        z1 = lax.dot_general(
            wc1_ref[...], hg, (((0,), (0,)), ((), ())), preferred_element_type=F32
        )
        z1 = jnp.maximum(z1 + bc1_ref[...], 0.0)
        z2 = lax.dot_general(
            wc2_ref[...], z1, (((0,), (0,)), ((), ())), preferred_element_type=F32
        )
        z2 = jnp.maximum(z2 + bc2_ref[...], 0.0)
        out = lax.dot_general(
            wc3_ref[...], z2, (((0,), (0,)), ((), ())), preferred_element_type=F32
        )
        out_ref[...] = out + bc3_ref[...]

    return pl.pallas_call(
        body,
        out_shape=jax.ShapeDtypeStruct((1, 1), F32),
    )(aggT, norm2, b2d, wc1, bc1, wc2, bc2, wc3, bc3)


def kernel(x, edge_index, W1, b1, W2, b2, Wc1, bc1, Wc2, bc2, Wc3, bc3):
    n, f = x.shape
    src = edge_index[0].astype(jnp.int32)
    dst = edge_index[1].astype(jnp.int32)

    deg_part = _degrees(src, dst, n)
    norm2 = _norms(deg_part, n)

    m1T = _mm1_scaled(x, W1, norm2)
    agg1T = _edge_pass(m1T, src, dst)
    m2T = _layer2_scaled(agg1T, W2, b1.reshape(f, 1), norm2)
    agg2T = _edge_pass(m2T, src, dst)
    out = _head(
        agg2T, norm2, b2.reshape(f, 1),
        Wc1, bc1.reshape(f, 1), Wc2, bc2.reshape(f, 1),
        Wc3, bc3.reshape(1, 1),
    )
    return out


# speedup vs baseline: 3.1769x; 1.2649x over previous
import functools

import jax
import jax.numpy as jnp
from jax import lax
from jax.experimental import pallas as pl
from jax.experimental.pallas import tpu as pltpu
from jax.experimental.pallas import tpu_sc as plsc

NC = 2
NS = 16
L = 16
NW = NC * NS

F32 = jnp.float32


def _mesh():
    return plsc.VectorSubcoreMesh(
        core_axis_name="c", subcore_axis_name="s", num_cores=NC, num_subcores=NS
    )


def _degrees(src, dst, n_nodes):
    e = src.shape[0]
    ept = e // NW

    @functools.partial(
        pl.kernel,
        out_type=jax.ShapeDtypeStruct((2 * NW, n_nodes), F32),
        mesh=_mesh(),
        compiler_params=pltpu.CompilerParams(needs_layout_passes=False),
        scratch_types=[
            pltpu.VMEM((ept,), jnp.int32),
            pltpu.VMEM((ept,), jnp.int32),
            pltpu.VMEM((n_nodes,), F32),
            pltpu.VMEM((n_nodes,), F32),
        ],
    )
    def deg_kernel(src_hbm, dst_hbm, out_hbm, s_v, d_v, od_v, id_v):
        wid = lax.axis_index("s") * NC + lax.axis_index("c")
        base = wid * ept
        pltpu.sync_copy(src_hbm.at[pl.ds(base, ept)], s_v)
        pltpu.sync_copy(dst_hbm.at[pl.ds(base, ept)], d_v)

        zero = jnp.zeros((L,), F32)

        def zbody(i, _):
            od_v[pl.ds(i * L, L)] = zero
            id_v[pl.ds(i * L, L)] = zero
            return ()

        lax.fori_loop(0, n_nodes // L, zbody, ())

        ones = jnp.ones((L,), F32)

        def ebody(g, _):
            s16 = s_v[pl.ds(g * L, L)]
            d16 = d_v[pl.ds(g * L, L)]
            plsc.addupdate_scatter(od_v, [s16], ones)
            plsc.addupdate_scatter(id_v, [d16], ones)
            return ()

        lax.fori_loop(0, ept // L, ebody, ())

        pltpu.sync_copy(od_v, out_hbm.at[wid])
        pltpu.sync_copy(id_v, out_hbm.at[NW + wid])

    return deg_kernel(src, dst)


def _norms(deg_part, n_nodes):
    def body(dp_ref, out_ref):
        od = jnp.sum(dp_ref[0:NW, :], axis=0, keepdims=True)
        idg = jnp.sum(dp_ref[NW : 2 * NW, :], axis=0, keepdims=True)
        ns = lax.rsqrt(jnp.maximum(od, 1.0))
        nd = lax.rsqrt(jnp.maximum(idg, 1.0))
        out_ref[...] = jnp.concatenate([ns, nd], axis=0)

    return pl.pallas_call(
        body,
        out_shape=jax.ShapeDtypeStruct((2, n_nodes), F32),
    )(deg_part)


def _mm1_scaled(x, w, norm2):
    n, f = x.shape

    def body(x_ref, w_ref, nrm_ref, out_ref):
        mm = lax.dot_general(
            w_ref[...], x_ref[...], (((0,), (1,)), ((), ())),
            preferred_element_type=F32,
        )
        out_ref[...] = mm * nrm_ref[0:1, :]

    return pl.pallas_call(
        body,
        out_shape=jax.ShapeDtypeStruct((f, n), F32),
    )(x, w, norm2)


def _layer2_scaled(aggT, w, b2d, norm2):
    f, n = aggT.shape

    def body(agg_ref, w_ref, b_ref, nrm_ref, out_ref):
        h = jnp.maximum(agg_ref[...] * nrm_ref[1:2, :] + b_ref[...], 0.0)
        mm = lax.dot_general(
            w_ref[...], h, (((0,), (0,)), ((), ())),
            preferred_element_type=F32,
        )
        out_ref[...] = mm * nrm_ref[0:1, :]

    return pl.pallas_call(
        body,
        out_shape=jax.ShapeDtypeStruct((f, n), F32),
    )(aggT, w, b2d, norm2)


def _edge_pass(mT, src, dst):
    f, n = mT.shape
    e = src.shape[0]
    fpt = f // NW
    chunk = 6400
    nch = e // chunk
    unroll = 8

    @functools.partial(
        pl.kernel,
        out_type=jax.ShapeDtypeStruct((f, n), F32),
        mesh=_mesh(),
        compiler_params=pltpu.CompilerParams(needs_layout_passes=False),
        scratch_types=[
            pltpu.VMEM((fpt, n), F32),
            pltpu.VMEM((fpt, n), F32),
            pltpu.VMEM((chunk,), jnp.int32),
            pltpu.VMEM((chunk,), jnp.int32),
            pltpu.VMEM((chunk,), jnp.int32),
            pltpu.VMEM((chunk,), jnp.int32),
            pltpu.SemaphoreType.DMA,
            pltpu.SemaphoreType.DMA,
            pltpu.SemaphoreType.DMA,
            pltpu.SemaphoreType.DMA,
        ],
    )
    def edge_kernel(mT_hbm, src_hbm, dst_hbm, aggT_hbm, m_v, agg_v,
                    s_a, d_a, s_b, d_b, sem_sa, sem_da, sem_sb, sem_db):
        wid = lax.axis_index("s") * NC + lax.axis_index("c")
        fbase = wid * fpt

        pltpu.async_copy(src_hbm.at[pl.ds(0, chunk)], s_a, sem_sa)
        pltpu.async_copy(dst_hbm.at[pl.ds(0, chunk)], d_a, sem_da)
        pltpu.sync_copy(mT_hbm.at[pl.ds(fbase, fpt), :], m_v)

        zero = jnp.zeros((L,), F32)

        def zbody(i, _):
            for j in range(fpt):
                agg_v[j, pl.ds(i * L, L)] = zero
            return ()

        lax.fori_loop(0, n // L, zbody, ())

        jidx = [jnp.full((L,), j, jnp.int32) for j in range(fpt)]

        def process(sbuf, dbuf):
            def ebody(g, _):
                base = g * (L * unroll)
                for u in range(unroll):
                    s16 = sbuf[pl.ds(base + u * L, L)]
                    d16 = dbuf[pl.ds(base + u * L, L)]
                    for j in range(fpt):
                        vals = plsc.load_gather(m_v, [jidx[j], s16])
                        plsc.addupdate_scatter(agg_v, [jidx[j], d16], vals)
                return ()

            lax.fori_loop(0, chunk // (L * unroll), ebody, ())

        def cbody(p, _):
            off_b = (2 * p + 1) * chunk
            pltpu.async_copy(src_hbm.at[pl.ds(off_b, chunk)], s_b, sem_sb)
            pltpu.async_copy(dst_hbm.at[pl.ds(off_b, chunk)], d_b, sem_db)
            pltpu.make_async_copy(src_hbm.at[pl.ds(0, chunk)], s_a, sem_sa).wait()
            pltpu.make_async_copy(dst_hbm.at[pl.ds(0, chunk)], d_a, sem_da).wait()
            process(s_a, d_a)

            @pl.when(p < nch // 2 - 1)
            def _():
                off_a = (2 * p + 2) * chunk
                pltpu.async_copy(src_hbm.at[pl.ds(off_a, chunk)], s_a, sem_sa)
                pltpu.async_copy(dst_hbm.at[pl.ds(off_a, chunk)], d_a, sem_da)

            pltpu.make_async_copy(src_hbm.at[pl.ds(0, chunk)], s_b, sem_sb).wait()
            pltpu.make_async_copy(dst_hbm.at[pl.ds(0, chunk)], d_b, sem_db).wait()
            process(s_b, d_b)
            return ()

        lax.fori_loop(0, nch // 2, cbody, ())

        pltpu.sync_copy(agg_v, aggT_hbm.at[pl.ds(fbase, fpt), :])

    return edge_kernel(mT, src, dst)


def _head(aggT, norm2, b2d, wc1, bc1, wc2, bc2, wc3, bc3):
    f, n = aggT.shape

    def body(agg_ref, nrm_ref, b_ref, wc1_ref, bc1_ref, wc2_ref, bc2_ref,
             wc3_ref, bc3_ref, out_ref):
        h = jnp.maximum(agg_ref[...] * nrm_ref[1:2, :] + b_ref[...], 0.0)
        hg = jnp.sum(h, axis=1, keepdims=True) * (1.0 / n)
        z1 = lax.dot_general(
            wc1_ref[...], hg, (((0,), (0,)), ((), ())), preferred_element_type=F32
        )
        z1 = jnp.maximum(z1 + bc1_ref[...], 0.0)
        z2 = lax.dot_general(
            wc2_ref[...], z1, (((0,), (0,)), ((), ())), preferred_element_type=F32
        )
        z2 = jnp.maximum(z2 + bc2_ref[...], 0.0)
        out = lax.dot_general(
            wc3_ref[...], z2, (((0,), (0,)), ((), ())), preferred_element_type=F32
        )
        out_ref[...] = out + bc3_ref[...]

    return pl.pallas_call(
        body,
        out_shape=jax.ShapeDtypeStruct((1, 1), F32),
    )(aggT, norm2, b2d, wc1, bc1, wc2, bc2, wc3, bc3)


def kernel(x, edge_index, W1, b1, W2, b2, Wc1, bc1, Wc2, bc2, Wc3, bc3):
    n, f = x.shape
    src = edge_index[0].astype(jnp.int32)
    dst = edge_index[1].astype(jnp.int32)

    deg_part = _degrees(src, dst, n)
    norm2 = _norms(deg_part, n)

    m1T = _mm1_scaled(x, W1, norm2)
    agg1T = _edge_pass(m1T, src, dst)
    m2T = _layer2_scaled(agg1T, W2, b1.reshape(f, 1), norm2)
    agg2T = _edge_pass(m2T, src, dst)
    out = _head(
        agg2T, norm2, b2.reshape(f, 1),
        Wc1, bc1.reshape(f, 1), Wc2, bc2.reshape(f, 1),
        Wc3, bc3.reshape(1, 1),
    )
    return out


# speedup vs baseline: 7.8821x; 2.4810x over previous
import functools

import jax
import jax.numpy as jnp
from jax import lax
from jax.experimental import pallas as pl
from jax.experimental.pallas import tpu as pltpu
from jax.experimental.pallas import tpu_sc as plsc

NC = 2
NS = 16
L = 16
NW = NC * NS

F32 = jnp.float32


def _mesh():
    return plsc.VectorSubcoreMesh(
        core_axis_name="c", subcore_axis_name="s", num_cores=NC, num_subcores=NS
    )


def _degrees(src, dst, n_nodes):
    e = src.shape[0]
    ept = e // NW

    @functools.partial(
        pl.kernel,
        out_type=jax.ShapeDtypeStruct((2 * NW, n_nodes), F32),
        mesh=_mesh(),
        compiler_params=pltpu.CompilerParams(needs_layout_passes=False),
        scratch_types=[
            pltpu.VMEM((ept,), jnp.int32),
            pltpu.VMEM((ept,), jnp.int32),
            pltpu.VMEM((n_nodes,), F32),
            pltpu.VMEM((n_nodes,), F32),
        ],
    )
    def deg_kernel(src_hbm, dst_hbm, out_hbm, s_v, d_v, od_v, id_v):
        wid = lax.axis_index("s") * NC + lax.axis_index("c")
        base = wid * ept
        pltpu.sync_copy(src_hbm.at[pl.ds(base, ept)], s_v)
        pltpu.sync_copy(dst_hbm.at[pl.ds(base, ept)], d_v)

        zero = jnp.zeros((L,), F32)

        def zbody(i, _):
            od_v[pl.ds(i * L, L)] = zero
            id_v[pl.ds(i * L, L)] = zero
            return ()

        lax.fori_loop(0, n_nodes // L, zbody, ())

        ones = jnp.ones((L,), F32)

        def ebody(g, _):
            s16 = s_v[pl.ds(g * L, L)]
            d16 = d_v[pl.ds(g * L, L)]
            plsc.addupdate_scatter(od_v, [s16], ones)
            plsc.addupdate_scatter(id_v, [d16], ones)
            return ()

        lax.fori_loop(0, ept // L, ebody, ())

        pltpu.sync_copy(od_v, out_hbm.at[wid])
        pltpu.sync_copy(id_v, out_hbm.at[NW + wid])

    return deg_kernel(src, dst)


def _norms(deg_part, n_nodes):
    def body(dp_ref, out_ref):
        od = jnp.sum(dp_ref[0:NW, :], axis=0, keepdims=True)
        idg = jnp.sum(dp_ref[NW : 2 * NW, :], axis=0, keepdims=True)
        ns = 1.0 / jnp.sqrt(jnp.maximum(od, 1.0))
        nd = 1.0 / jnp.sqrt(jnp.maximum(idg, 1.0))
        out_ref[...] = jnp.concatenate([ns, nd], axis=0)

    return pl.pallas_call(
        body,
        out_shape=jax.ShapeDtypeStruct((2, n_nodes), F32),
    )(deg_part)


def _mm1_scaled(x, w, norm2):
    n, f = x.shape

    def body(x_ref, w_ref, nrm_ref, out_ref):
        mm = lax.dot_general(
            w_ref[...], x_ref[...], (((0,), (1,)), ((), ())),
            preferred_element_type=F32,
        )
        out_ref[...] = mm * nrm_ref[0:1, :]

    return pl.pallas_call(
        body,
        out_shape=jax.ShapeDtypeStruct((f, n), F32),
    )(x, w, norm2)


def _layer2_scaled(aggT, w, b2d, norm2):
    f, n = aggT.shape

    def body(agg_ref, w_ref, b_ref, nrm_ref, out_ref):
        h = jnp.maximum(agg_ref[...] * nrm_ref[1:2, :] + b_ref[...], 0.0)
        mm = lax.dot_general(
            w_ref[...], h, (((0,), (0,)), ((), ())),
            preferred_element_type=F32,
        )
        out_ref[...] = mm * nrm_ref[0:1, :]

    return pl.pallas_call(
        body,
        out_shape=jax.ShapeDtypeStruct((f, n), F32),
    )(aggT, w, b2d, norm2)


def _edge_pass(mT, src, dst):
    f, n = mT.shape
    e = src.shape[0]
    fpt = f // NW
    chunk = 6400
    nch = e // chunk
    unroll = 16

    @functools.partial(
        pl.kernel,
        out_type=jax.ShapeDtypeStruct((f, n), F32),
        mesh=_mesh(),
        compiler_params=pltpu.CompilerParams(needs_layout_passes=False),
        scratch_types=(
            [pltpu.VMEM((n,), F32) for _ in range(2 * fpt)]
            + [
                pltpu.VMEM((chunk,), jnp.int32),
                pltpu.VMEM((chunk,), jnp.int32),
                pltpu.VMEM((chunk,), jnp.int32),
                pltpu.VMEM((chunk,), jnp.int32),
                pltpu.SemaphoreType.DMA,
                pltpu.SemaphoreType.DMA,
                pltpu.SemaphoreType.DMA,
                pltpu.SemaphoreType.DMA,
            ]
        ),
    )
    def edge_kernel(mT_hbm, src_hbm, dst_hbm, aggT_hbm, *rest):
        m_vs = rest[:fpt]
        agg_vs = rest[fpt : 2 * fpt]
        s_a, d_a, s_b, d_b, sem_sa, sem_da, sem_sb, sem_db = rest[2 * fpt :]
        wid = lax.axis_index("s") * NC + lax.axis_index("c")
        fbase = wid * fpt

        pltpu.async_copy(src_hbm.at[pl.ds(0, chunk)], s_a, sem_sa)
        pltpu.async_copy(dst_hbm.at[pl.ds(0, chunk)], d_a, sem_da)
        for j in range(fpt):
            pltpu.sync_copy(mT_hbm.at[fbase + j], m_vs[j])

        zero = jnp.zeros((L,), F32)

        def zbody(i, _):
            for j in range(fpt):
                agg_vs[j][pl.ds(i * L, L)] = zero
            return ()

        lax.fori_loop(0, n // L, zbody, ())

        def process(sbuf, dbuf):
            @plsc.parallel_loop(0, chunk // L, 1, unroll=unroll)
            def _(g):
                s16 = sbuf[pl.ds(g * L, L)]
                d16 = dbuf[pl.ds(g * L, L)]
                for j in range(fpt):
                    vals = plsc.load_gather(m_vs[j], [s16])
                    plsc.addupdate_scatter(agg_vs[j], [d16], vals)

        def cbody(p, _):
            off_b = (2 * p + 1) * chunk
            pltpu.async_copy(src_hbm.at[pl.ds(off_b, chunk)], s_b, sem_sb)
            pltpu.async_copy(dst_hbm.at[pl.ds(off_b, chunk)], d_b, sem_db)
            pltpu.make_async_copy(src_hbm.at[pl.ds(0, chunk)], s_a, sem_sa).wait()
            pltpu.make_async_copy(dst_hbm.at[pl.ds(0, chunk)], d_a, sem_da).wait()
            process(s_a, d_a)

            @pl.when(p < nch // 2 - 1)
            def _():
                off_a = (2 * p + 2) * chunk
                pltpu.async_copy(src_hbm.at[pl.ds(off_a, chunk)], s_a, sem_sa)
                pltpu.async_copy(dst_hbm.at[pl.ds(off_a, chunk)], d_a, sem_da)

            pltpu.make_async_copy(src_hbm.at[pl.ds(0, chunk)], s_b, sem_sb).wait()
            pltpu.make_async_copy(dst_hbm.at[pl.ds(0, chunk)], d_b, sem_db).wait()
            process(s_b, d_b)
            return ()

        lax.fori_loop(0, nch // 2, cbody, ())

        for j in range(fpt):
            pltpu.sync_copy(agg_vs[j], aggT_hbm.at[fbase + j])

    return edge_kernel(mT, src, dst)


def _head(aggT, norm2, b2d, wc1, bc1, wc2, bc2, wc3, bc3):
    f, n = aggT.shape

    def body(agg_ref, nrm_ref, b_ref, wc1_ref, bc1_ref, wc2_ref, bc2_ref,
             wc3_ref, bc3_ref, out_ref):
        h = jnp.maximum(agg_ref[...] * nrm_ref[1:2, :] + b_ref[...], 0.0)
        hg = jnp.sum(h, axis=1, keepdims=True) * (1.0 / n)
        z1 = lax.dot_general(
            wc1_ref[...], hg, (((0,), (0,)), ((), ())), preferred_element_type=F32
        )
        z1 = jnp.maximum(z1 + bc1_ref[...], 0.0)
        z2 = lax.dot_general(
            wc2_ref[...], z1, (((0,), (0,)), ((), ())), preferred_element_type=F32
        )
        z2 = jnp.maximum(z2 + bc2_ref[...], 0.0)
        out = lax.dot_general(
            wc3_ref[...], z2, (((0,), (0,)), ((), ())), preferred_element_type=F32
        )
        out_ref[...] = out + bc3_ref[...]

    return pl.pallas_call(
        body,
        out_shape=jax.ShapeDtypeStruct((1, 1), F32),
    )(aggT, norm2, b2d, wc1, bc1, wc2, bc2, wc3, bc3)


def kernel(x, edge_index, W1, b1, W2, b2, Wc1, bc1, Wc2, bc2, Wc3, bc3):
    n, f = x.shape
    src = edge_index[0].astype(jnp.int32)
    dst = edge_index[1].astype(jnp.int32)

    deg_part = _degrees(src, dst, n)
    norm2 = _norms(deg_part, n)

    m1T = _mm1_scaled(x, W1, norm2)
    agg1T = _edge_pass(m1T, src, dst)
    m2T = _layer2_scaled(agg1T, W2, b1.reshape(f, 1), norm2)
    agg2T = _edge_pass(m2T, src, dst)
    out = _head(
        agg2T, norm2, b2.reshape(f, 1),
        Wc1, bc1.reshape(f, 1), Wc2, bc2.reshape(f, 1),
        Wc3, bc3.reshape(1, 1),
    )
    return out
